# baseline (speedup 1.0000x reference)
import functools

import jax
import jax.numpy as jnp
from jax import lax
from jax.experimental import pallas as pl
from jax.experimental.pallas import tpu as pltpu
from jax.experimental.pallas import tpu_sc as plsc

_NC = 2
_NS = 16
_LANES = 16
_CHUNK = 96


def _tc_linear_body(a0_ref, a1_ref, w0_ref, w1_ref, b_ref, o_ref):
    h = jnp.dot(a0_ref[0], w0_ref[...], preferred_element_type=jnp.float32)
    h = h + jnp.dot(a1_ref[0], w1_ref[...], preferred_element_type=jnp.float32)
    o_ref[...] = jnp.tanh(h + b_ref[...])


def _make_sc_aggregate(n, dh, kc, rows_per_tile):
    rows_sh = _NS * rows_per_tile
    mesh = plsc.VectorSubcoreMesh(core_axis_name="c", subcore_axis_name="s")

    @functools.partial(
        pl.kernel,
        out_type=jax.ShapeDtypeStruct((_NC, rows_sh, dh), jnp.float32),
        mesh=mesh,
        scratch_types=[
            pltpu.VMEM((kc * _CHUNK,), jnp.int32),
            pltpu.VMEM((kc, _CHUNK), jnp.int32),
            [pltpu.VMEM((_CHUNK, dh), jnp.float32) for _ in range(2)],
            pltpu.VMEM_SHARED((rows_sh, dh), jnp.float32),
            [pltpu.SemaphoreType.DMA for _ in range(2)],
            [pltpu.SemaphoreType.DMA for _ in range(2)],
            pltpu.SemaphoreType.DMA,
        ],
    )
    def agg_kernel(x_hbm, src_hbm, dst_hbm, z_hbm, out_hbm, sidx, didx, rows, acc,
                   gsem, ssem, zsem):
        c = lax.axis_index("c")
        s = lax.axis_index("s")
        zbase = s * rows_per_tile

        def start_gather(j, p):
            pltpu.async_copy(x_hbm.at[sidx.at[pl.ds(j * _CHUNK, _CHUNK)]], rows[p], gsem[p])

        def wait_gather(j, p):
            pltpu.make_async_copy(x_hbm.at[sidx.at[pl.ds(j * _CHUNK, _CHUNK)]], rows[p], gsem[p]).wait()

        def start_scatter(j, p):
            pltpu.async_copy(rows[p], acc.at[didx.at[j]], ssem[p], add=True)

        def wait_scatter(j, p):
            pltpu.make_async_copy(rows[p], acc.at[didx.at[j]], ssem[p]).wait()

        pltpu.async_copy(z_hbm, acc.at[pl.ds(zbase, rows_per_tile)], zsem)
        pltpu.sync_copy(src_hbm.at[c, s], sidx)
        pltpu.sync_copy(dst_hbm.at[s], didx)
        start_gather(0, 0)
        start_gather(1, 1)
        pltpu.make_async_copy(z_hbm, acc.at[pl.ds(zbase, rows_per_tile)], zsem).wait()
        plsc.subcore_barrier()

        def pair_body(m, carry):
            j = 2 * m
            wait_gather(j, 0)
            start_scatter(j, 0)
            wait_gather(j + 1, 1)
            start_scatter(j + 1, 1)
            wait_scatter(j, 0)
            start_gather(j + 2, 0)
            wait_scatter(j + 1, 1)
            start_gather(j + 3, 1)
            return carry

        lax.fori_loop(0, kc // 2 - 1, pair_body, 0)

        wait_gather(kc - 2, 0)
        start_scatter(kc - 2, 0)
        wait_gather(kc - 1, 1)
        start_scatter(kc - 1, 1)
        wait_scatter(kc - 2, 0)
        wait_scatter(kc - 1, 1)

        plsc.subcore_barrier()

        rbase = s * rows_per_tile
        off = 0
        while off < rows_per_tile:
            w = min(_CHUNK, rows_per_tile - off)
            pltpu.sync_copy(acc.at[pl.ds(rbase + off, w)], rows[0].at[pl.ds(0, w)])
            pltpu.sync_copy(rows[0].at[pl.ds(0, w)], out_hbm.at[c, pl.ds(rbase + off, w)])
            off += w

    return agg_kernel


def kernel(x, edge_index, W, b):
    n, d = x.shape
    e = edge_index.shape[1]
    dh = d // 2

    src = edge_index[0].astype(jnp.int32)
    dst = edge_index[1].astype(jnp.int32)

    epb = _NS * _CHUNK
    kc = 2 * -(-e // (2 * epb))
    e_pad = kc * epb
    pad = e_pad - e
    if pad:
        src = jnp.concatenate([src, jnp.zeros((pad,), jnp.int32)])
        dst = jnp.concatenate([dst, jnp.full((pad,), n, jnp.int32)])

    xflat = x.reshape(n * 2, dh)
    src2 = jnp.stack([2 * src, 2 * src + 1]).reshape(_NC, _NS, kc * _CHUNK)
    dst3 = dst.reshape(_NS, kc, _CHUNK)

    rows_per_tile = -(-(-(-(n + 1) // _NS)) // 8) * 8
    zeros = jnp.zeros((rows_per_tile, dh), jnp.float32)

    agg3 = _make_sc_aggregate(n, dh, kc, rows_per_tile)(xflat, src2, dst3, zeros)

    rblk = 1000
    tc = pl.pallas_call(
        _tc_linear_body,
        grid=(n // rblk,),
        in_specs=[
            pl.BlockSpec((1, rblk, dh), lambda i: (0, i, 0)),
            pl.BlockSpec((1, rblk, dh), lambda i: (1, i, 0)),
            pl.BlockSpec((dh, d), lambda i: (0, 0)),
            pl.BlockSpec((dh, d), lambda i: (0, 0)),
            pl.BlockSpec((1, d), lambda i: (0, 0)),
        ],
        out_specs=pl.BlockSpec((rblk, d), lambda i: (i, 0)),
        out_shape=jax.ShapeDtypeStruct((n, d), jnp.float32),
    )
    wt = W.T
    return tc(agg3, agg3, wt[:dh], wt[dh:], b.reshape(1, d))


# speedup vs baseline: 1.1055x; 1.1055x over previous
import functools

import jax
import jax.numpy as jnp
from jax import lax
from jax.experimental import pallas as pl
from jax.experimental.pallas import tpu as pltpu
from jax.experimental.pallas import tpu_sc as plsc

_NC = 2
_NS = 16
_LANES = 16
_CHUNK = 96


def _tc_linear_body(a0_ref, a1_ref, w0_ref, w1_ref, b_ref, o_ref):
    h = jnp.dot(a0_ref[0], w0_ref[...], preferred_element_type=jnp.float32)
    h = h + jnp.dot(a1_ref[0], w1_ref[...], preferred_element_type=jnp.float32)
    o_ref[...] = jnp.tanh(h + b_ref[...])


def _make_sc_aggregate(n, dh, kc, rows_per_tile):
    rows_sh = _NS * rows_per_tile
    mesh = plsc.VectorSubcoreMesh(core_axis_name="c", subcore_axis_name="s")

    @functools.partial(
        pl.kernel,
        out_type=jax.ShapeDtypeStruct((_NC, rows_sh, dh), jnp.float32),
        mesh=mesh,
        scratch_types=[
            pltpu.VMEM((kc * _CHUNK,), jnp.int32),
            pltpu.VMEM((kc, _CHUNK), jnp.int32),
            [pltpu.VMEM((_CHUNK, dh), jnp.float32) for _ in range(2)],
            pltpu.VMEM_SHARED((rows_sh, dh), jnp.float32),
            [pltpu.SemaphoreType.DMA for _ in range(2)],
            [pltpu.SemaphoreType.DMA for _ in range(2)],
            pltpu.SemaphoreType.DMA,
        ],
    )
    def agg_kernel(x_hbm, src_hbm, dst_hbm, z_hbm, out_hbm, sidx, didx, rows, acc,
                   gsem, ssem, zsem):
        c = lax.axis_index("c")
        s = lax.axis_index("s")
        zbase = s * rows_per_tile

        def start_gather(j, p):
            pltpu.async_copy(x_hbm.at[sidx.at[pl.ds(j * _CHUNK, _CHUNK)]], rows[p], gsem[p])

        def wait_gather(j, p):
            pltpu.make_async_copy(x_hbm.at[sidx.at[pl.ds(j * _CHUNK, _CHUNK)]], rows[p], gsem[p]).wait()

        def start_scatter(j, p):
            pltpu.async_copy(rows[p], acc.at[didx.at[j]], ssem[p], add=True)

        def wait_scatter(j, p):
            pltpu.make_async_copy(rows[p], acc.at[didx.at[j]], ssem[p]).wait()

        pltpu.async_copy(z_hbm, acc.at[pl.ds(zbase, rows_per_tile)], zsem)
        pltpu.sync_copy(src_hbm.at[c, s], sidx)
        pltpu.sync_copy(dst_hbm.at[s], didx)
        start_gather(0, 0)
        start_gather(1, 1)
        pltpu.make_async_copy(z_hbm, acc.at[pl.ds(zbase, rows_per_tile)], zsem).wait()
        plsc.subcore_barrier()

        def scatter(j, p):
            pltpu.sync_copy(rows[p], acc.at[didx.at[j]], add=True)

        def pair_body(m, carry):
            j = 2 * m
            wait_gather(j, 0)
            scatter(j, 0)
            start_gather(j + 2, 0)
            wait_gather(j + 1, 1)
            scatter(j + 1, 1)
            start_gather(j + 3, 1)
            return carry

        lax.fori_loop(0, kc // 2 - 1, pair_body, 0)

        wait_gather(kc - 2, 0)
        scatter(kc - 2, 0)
        wait_gather(kc - 1, 1)
        scatter(kc - 1, 1)

        plsc.subcore_barrier()

        rbase = s * rows_per_tile
        off = 0
        while off < rows_per_tile:
            w = min(_CHUNK, rows_per_tile - off)
            pltpu.sync_copy(acc.at[pl.ds(rbase + off, w)], rows[0].at[pl.ds(0, w)])
            pltpu.sync_copy(rows[0].at[pl.ds(0, w)], out_hbm.at[c, pl.ds(rbase + off, w)])
            off += w

    return agg_kernel


def kernel(x, edge_index, W, b):
    n, d = x.shape
    e = edge_index.shape[1]
    dh = d // 2

    src = edge_index[0].astype(jnp.int32)
    dst = edge_index[1].astype(jnp.int32)

    epb = _NS * _CHUNK
    kc = 2 * -(-e // (2 * epb))
    e_pad = kc * epb
    pad = e_pad - e
    if pad:
        src = jnp.concatenate([src, jnp.zeros((pad,), jnp.int32)])
        dst = jnp.concatenate([dst, jnp.full((pad,), n, jnp.int32)])

    xflat = x.reshape(n * 2, dh)
    src2 = jnp.stack([2 * src, 2 * src + 1]).reshape(_NC, _NS, kc * _CHUNK)
    dst3 = dst.reshape(_NS, kc, _CHUNK)

    rows_per_tile = -(-(-(-(n + 1) // _NS)) // 8) * 8
    zeros = jnp.zeros((rows_per_tile, dh), jnp.float32)

    agg3 = _make_sc_aggregate(n, dh, kc, rows_per_tile)(xflat, src2, dst3, zeros)

    rblk = 1000
    tc = pl.pallas_call(
        _tc_linear_body,
        grid=(n // rblk,),
        in_specs=[
            pl.BlockSpec((1, rblk, dh), lambda i: (0, i, 0)),
            pl.BlockSpec((1, rblk, dh), lambda i: (1, i, 0)),
            pl.BlockSpec((dh, d), lambda i: (0, 0)),
            pl.BlockSpec((dh, d), lambda i: (0, 0)),
            pl.BlockSpec((1, d), lambda i: (0, 0)),
        ],
        out_specs=pl.BlockSpec((rblk, d), lambda i: (i, 0)),
        out_shape=jax.ShapeDtypeStruct((n, d), jnp.float32),
    )
    wt = W.T
    return tc(agg3, agg3, wt[:dh], wt[dh:], b.reshape(1, d))


# speedup vs baseline: 1.1092x; 1.0034x over previous
import functools

import jax
import jax.numpy as jnp
from jax import lax
from jax.experimental import pallas as pl
from jax.experimental.pallas import tpu as pltpu
from jax.experimental.pallas import tpu_sc as plsc

_NC = 2
_NS = 16
_LANES = 16
_CHUNK = 96


def _tc_linear_body(a0_ref, a1_ref, w0_ref, w1_ref, b_ref, o_ref):
    h = jnp.dot(a0_ref[0], w0_ref[...], preferred_element_type=jnp.float32)
    h = h + jnp.dot(a1_ref[0], w1_ref[...], preferred_element_type=jnp.float32)
    o_ref[...] = jnp.tanh(h + b_ref[...])


def _make_sc_aggregate(n, dh, kc, rows_per_tile):
    rows_sh = _NS * rows_per_tile
    mesh = plsc.VectorSubcoreMesh(core_axis_name="c", subcore_axis_name="s")

    @functools.partial(
        pl.kernel,
        out_type=jax.ShapeDtypeStruct((_NC, rows_sh, dh), jnp.float32),
        mesh=mesh,
        scratch_types=[
            pltpu.VMEM((kc * _CHUNK,), jnp.int32),
            pltpu.VMEM((kc, _CHUNK), jnp.int32),
            [pltpu.VMEM((_CHUNK, dh), jnp.float32) for _ in range(2)],
            pltpu.VMEM_SHARED((rows_sh, dh), jnp.float32),
            [pltpu.SemaphoreType.DMA for _ in range(2)],
            [pltpu.SemaphoreType.DMA for _ in range(2)],
            pltpu.SemaphoreType.DMA,
        ],
    )
    def agg_kernel(x_hbm, src_hbm, dst_hbm, z_hbm, out_hbm, sidx, didx, rows, acc,
                   gsem, ssem, zsem):
        c = lax.axis_index("c")
        s = lax.axis_index("s")
        zbase = s * rows_per_tile

        def start_gather(j, p):
            pltpu.async_copy(x_hbm.at[sidx.at[pl.ds(j * _CHUNK, _CHUNK)]], rows[p], gsem[p])

        def wait_gather(j, p):
            pltpu.make_async_copy(x_hbm.at[sidx.at[pl.ds(j * _CHUNK, _CHUNK)]], rows[p], gsem[p]).wait()

        def start_scatter(j, p):
            pltpu.async_copy(rows[p], acc.at[didx.at[j]], ssem[p], add=True)

        def wait_scatter(j, p):
            pltpu.make_async_copy(rows[p], acc.at[didx.at[j]], ssem[p]).wait()

        pltpu.async_copy(z_hbm, acc.at[pl.ds(zbase, rows_per_tile)], zsem)
        pltpu.sync_copy(src_hbm.at[c, s], sidx)
        pltpu.sync_copy(dst_hbm.at[s], didx)
        start_gather(0, 0)
        start_gather(1, 1)
        pltpu.make_async_copy(z_hbm, acc.at[pl.ds(zbase, rows_per_tile)], zsem).wait()
        plsc.subcore_barrier()

        def scatter(j, p):
            pltpu.sync_copy(rows[p], acc.at[didx.at[j]], add=True)

        def pair_body(m, carry):
            j = 2 * m
            wait_gather(j, 0)
            scatter(j, 0)
            start_gather(j + 2, 0)
            wait_gather(j + 1, 1)
            scatter(j + 1, 1)
            start_gather(j + 3, 1)
            return carry

        lax.fori_loop(0, kc // 2 - 1, pair_body, 0)

        wait_gather(kc - 2, 0)
        scatter(kc - 2, 0)
        wait_gather(kc - 1, 1)
        scatter(kc - 1, 1)

        plsc.subcore_barrier()

        rbase = s * rows_per_tile
        pltpu.sync_copy(acc.at[pl.ds(rbase, rows_per_tile)],
                        out_hbm.at[c, pl.ds(rbase, rows_per_tile)])

    return agg_kernel


def kernel(x, edge_index, W, b):
    n, d = x.shape
    e = edge_index.shape[1]
    dh = d // 2

    src = edge_index[0].astype(jnp.int32)
    dst = edge_index[1].astype(jnp.int32)

    epb = _NS * _CHUNK
    kc = 2 * -(-e // (2 * epb))
    e_pad = kc * epb
    pad = e_pad - e
    if pad:
        src = jnp.concatenate([src, jnp.zeros((pad,), jnp.int32)])
        dst = jnp.concatenate([dst, jnp.full((pad,), n, jnp.int32)])

    xflat = x.reshape(n * 2, dh)
    src2 = jnp.stack([2 * src, 2 * src + 1]).reshape(_NC, _NS, kc * _CHUNK)
    dst3 = dst.reshape(_NS, kc, _CHUNK)

    rows_per_tile = -(-(-(-(n + 1) // _NS)) // 8) * 8
    zeros = jnp.zeros((rows_per_tile, dh), jnp.float32)

    agg3 = _make_sc_aggregate(n, dh, kc, rows_per_tile)(xflat, src2, dst3, zeros)

    rblk = 1000
    tc = pl.pallas_call(
        _tc_linear_body,
        grid=(n // rblk,),
        in_specs=[
            pl.BlockSpec((1, rblk, dh), lambda i: (0, i, 0)),
            pl.BlockSpec((1, rblk, dh), lambda i: (1, i, 0)),
            pl.BlockSpec((dh, d), lambda i: (0, 0)),
            pl.BlockSpec((dh, d), lambda i: (0, 0)),
            pl.BlockSpec((1, d), lambda i: (0, 0)),
        ],
        out_specs=pl.BlockSpec((rblk, d), lambda i: (i, 0)),
        out_shape=jax.ShapeDtypeStruct((n, d), jnp.float32),
    )
    wt = W.T
    return tc(agg3, agg3, wt[:dh], wt[dh:], b.reshape(1, d))
